# Initial kernel scaffold; baseline (speedup 1.0000x reference)
import functools

import jax
import jax.numpy as jnp
from jax import lax
from jax.experimental import pallas as pl
from jax.experimental.pallas import tpu as pltpu
from jax.experimental.pallas import tpu_sc as plsc

NC = 2
NS = 16
NW = NC * NS


def _sc_mesh():
    return plsc.VectorSubcoreMesh(
        core_axis_name="c", subcore_axis_name="s",
        num_cores=NC, num_subcores=NS)


def _make_deg_kernel(n, e, chunk):
    iters = e // NW // chunk
    rpt = n // NS

    @functools.partial(
        pl.kernel,
        out_type=jax.ShapeDtypeStruct((NC, n, 16), jnp.float32),
        mesh=_sc_mesh(),
        scratch_types=[
            pltpu.VMEM_SHARED((n, 16), jnp.float32),
            pltpu.VMEM((chunk,), jnp.int32),
            pltpu.VMEM((chunk, 16), jnp.float32),
        ],
    )
    def deg_kernel(dst_hbm, zeros_hbm, ones_hbm, out_hbm, deg_sh, idx_v, ones_v):
        c = lax.axis_index("c")
        s = lax.axis_index("s")
        pltpu.sync_copy(zeros_hbm, deg_sh.at[pl.ds(s * rpt, rpt)])
        pltpu.sync_copy(ones_hbm, ones_v)
        plsc.subcore_barrier()
        base = (c * NS + s) * (e // NW)

        def step(i, carry):
            pltpu.sync_copy(dst_hbm.at[pl.ds(base + i * chunk, chunk)], idx_v)
            pltpu.sync_copy(ones_v, deg_sh.at[idx_v], add=True)
            return carry

        lax.fori_loop(0, iters, step, 0)
        plsc.subcore_barrier()
        pltpu.sync_copy(deg_sh.at[pl.ds(s * rpt, rpt)],
                        out_hbm.at[c, pl.ds(s * rpt, rpt)])

    return deg_kernel


def _make_edge_kernel(n, d, e, chunk):
    iters = e // NW // chunk
    rpt = n // NS

    @functools.partial(
        pl.kernel,
        out_type=jax.ShapeDtypeStruct((NC, n, d), jnp.float32),
        mesh=_sc_mesh(),
        scratch_types=[
            pltpu.VMEM_SHARED((n, d), jnp.float32),
            pltpu.VMEM((chunk,), jnp.int32),
            pltpu.VMEM((chunk,), jnp.int32),
            pltpu.VMEM((chunk, d), jnp.float32),
            pltpu.SemaphoreType.DMA,
        ],
    )
    def edge_kernel(h_hbm, src_hbm, dst_hbm, zeros_hbm, out_hbm,
                    acc_sh, src_v, dst_v, rows_v, sem):
        c = lax.axis_index("c")
        s = lax.axis_index("s")
        pltpu.sync_copy(zeros_hbm, acc_sh.at[pl.ds(s * rpt, rpt)])
        plsc.subcore_barrier()
        base = (c * NS + s) * (e // NW)

        def step(i, carry):
            off = base + i * chunk
            pltpu.sync_copy(src_hbm.at[pl.ds(off, chunk)], src_v)
            pltpu.sync_copy(dst_hbm.at[pl.ds(off, chunk)], dst_v)
            pltpu.async_copy(h_hbm.at[src_v], rows_v, sem).wait()
            pltpu.sync_copy(rows_v, acc_sh.at[dst_v], add=True)
            return carry

        lax.fori_loop(0, iters, step, 0)
        plsc.subcore_barrier()
        pltpu.sync_copy(acc_sh.at[pl.ds(s * rpt, rpt)],
                        out_hbm.at[c, pl.ds(s * rpt, rpt)])

    return edge_kernel


def _dense_pre(degp, x, W1, bn):
    n, d = x.shape

    def body(degp_ref, x_ref, w_ref, h_ref, dinv_ref):
        p = degp_ref[...]
        dv = lax.rsqrt(1.0 + p[0, :, :1] + p[1, :, :1])
        h = jnp.dot(x_ref[...], w_ref[...], preferred_element_type=jnp.float32)
        h_ref[...] = h * dv
        dinv_ref[...] = dv

    return pl.pallas_call(
        body,
        grid=(n // bn,),
        in_specs=[
            pl.BlockSpec((NC, bn, 16), lambda i: (0, i, 0)),
            pl.BlockSpec((bn, d), lambda i: (i, 0)),
            pl.BlockSpec((d, d), lambda i: (0, 0)),
        ],
        out_specs=[
            pl.BlockSpec((bn, d), lambda i: (i, 0)),
            pl.BlockSpec((bn, 1), lambda i: (i, 0)),
        ],
        out_shape=[
            jax.ShapeDtypeStruct((n, d), jnp.float32),
            jax.ShapeDtypeStruct((n, 1), jnp.float32),
        ],
    )(degp, x, W1)


def _dense_mid(accp, h1s, dinv, b1, W2, bn):
    n, d = h1s.shape

    def body(accp_ref, h_ref, dinv_ref, b_ref, w_ref, o_ref):
        p = accp_ref[...]
        dv = dinv_ref[...]
        z = (p[0] + p[1] + h_ref[...]) * dv + b_ref[...]
        g = jax.nn.gelu(z)
        o_ref[...] = jnp.dot(g, w_ref[...],
                             preferred_element_type=jnp.float32) * dv

    return pl.pallas_call(
        body,
        grid=(n // bn,),
        in_specs=[
            pl.BlockSpec((NC, bn, d), lambda i: (0, i, 0)),
            pl.BlockSpec((bn, d), lambda i: (i, 0)),
            pl.BlockSpec((bn, 1), lambda i: (i, 0)),
            pl.BlockSpec((1, d), lambda i: (0, 0)),
            pl.BlockSpec((d, d), lambda i: (0, 0)),
        ],
        out_specs=pl.BlockSpec((bn, d), lambda i: (i, 0)),
        out_shape=jax.ShapeDtypeStruct((n, d), jnp.float32),
    )(accp, h1s, dinv, b1, W2)


def _dense_post(accp, h2s, dinv, b2, bn):
    n, d = h2s.shape

    def body(accp_ref, h_ref, dinv_ref, b_ref, o_ref):
        p = accp_ref[...]
        o_ref[...] = (p[0] + p[1] + h_ref[...]) * dinv_ref[...] + b_ref[...]

    return pl.pallas_call(
        body,
        grid=(n // bn,),
        in_specs=[
            pl.BlockSpec((NC, bn, d), lambda i: (0, i, 0)),
            pl.BlockSpec((bn, d), lambda i: (i, 0)),
            pl.BlockSpec((bn, 1), lambda i: (i, 0)),
            pl.BlockSpec((1, d), lambda i: (0, 0)),
        ],
        out_specs=pl.BlockSpec((bn, d), lambda i: (i, 0)),
        out_shape=jax.ShapeDtypeStruct((n, d), jnp.float32),
    )(accp, h2s, dinv, b2)


def kernel(x, edge_index, W1, b1, W2, b2):
    n, d = x.shape
    e = edge_index.shape[1]
    chunk = 80
    bn = 1000
    assert e % (NW * chunk) == 0 and n % NS == 0 and n % bn == 0

    src = edge_index[0].astype(jnp.int32)
    dst = edge_index[1].astype(jnp.int32)
    zeros16 = jnp.zeros((n // NS, 16), jnp.float32)
    ones16 = jnp.ones((chunk, 16), jnp.float32)
    zerosd = jnp.zeros((n // NS, d), jnp.float32)
    b1r = b1.reshape(1, d)
    b2r = b2.reshape(1, d)

    edge_k = _make_edge_kernel(n, d, e, chunk)

    degp = _make_deg_kernel(n, e, chunk)(dst, zeros16, ones16)
    h1s, dinv = _dense_pre(degp, x, W1, bn)
    acc1 = edge_k(h1s, src, dst, zerosd)
    h2s = _dense_mid(acc1, h1s, dinv, b1r, W2, bn)
    acc2 = edge_k(h2s, src, dst, zerosd)
    return _dense_post(acc2, h2s, dinv, b2r, bn)


# speedup vs baseline: 12.2604x; 12.2604x over previous
import functools

import jax
import jax.numpy as jnp
from jax import lax
from jax.experimental import pallas as pl
from jax.experimental.pallas import tpu as pltpu
from jax.experimental.pallas import tpu_sc as plsc

NC = 2
NS = 16
NW = NC * NS


def _sc_mesh():
    return plsc.VectorSubcoreMesh(
        core_axis_name="c", subcore_axis_name="s",
        num_cores=NC, num_subcores=NS)


def _pad_nodes(n):
    return ((n + NS * 8 - 1) // (NS * 8)) * (NS * 8)


def _make_deg_kernel(n, e, chunk):
    iters = e // NW // chunk
    np_ = _pad_nodes(n)
    rpt = np_ // NS

    @functools.partial(
        pl.kernel,
        out_type=jax.ShapeDtypeStruct((NC, np_, 128), jnp.float32),
        mesh=_sc_mesh(),
        scratch_types=[
            pltpu.VMEM_SHARED((np_, 128), jnp.float32),
            pltpu.VMEM((chunk,), jnp.int32),
            pltpu.VMEM((chunk, 128), jnp.float32),
        ],
    )
    def deg_kernel(dst_hbm, zeros_hbm, ones_hbm, out_hbm, deg_sh, idx_v, ones_v):
        c = lax.axis_index("c")
        s = lax.axis_index("s")
        pltpu.sync_copy(zeros_hbm, deg_sh.at[pl.ds(s * rpt, rpt)])
        pltpu.sync_copy(ones_hbm, ones_v)
        plsc.subcore_barrier()
        base = (c * NS + s) * (e // NW)

        def step(i, carry):
            pltpu.sync_copy(dst_hbm.at[pl.ds(base + i * chunk, chunk)], idx_v)
            pltpu.sync_copy(ones_v, deg_sh.at[idx_v], add=True)
            return carry

        lax.fori_loop(0, iters, step, 0)
        plsc.subcore_barrier()
        pltpu.sync_copy(deg_sh.at[pl.ds(s * rpt, rpt)],
                        out_hbm.at[c, pl.ds(s * rpt, rpt)])

    return deg_kernel


def _make_edge_kernel(n, d, e, chunk):
    iters = e // NW // chunk
    np_ = _pad_nodes(n)
    rpt = np_ // NS

    @functools.partial(
        pl.kernel,
        out_type=jax.ShapeDtypeStruct((NC, np_, d), jnp.float32),
        mesh=_sc_mesh(),
        scratch_types=[
            pltpu.VMEM_SHARED((np_, d), jnp.float32),
            pltpu.VMEM((chunk,), jnp.int32),
            pltpu.VMEM((chunk,), jnp.int32),
            pltpu.VMEM((chunk, d), jnp.float32),
            pltpu.SemaphoreType.DMA,
        ],
    )
    def edge_kernel(h_hbm, src_hbm, dst_hbm, zeros_hbm, out_hbm,
                    acc_sh, src_v, dst_v, rows_v, sem):
        c = lax.axis_index("c")
        s = lax.axis_index("s")
        pltpu.sync_copy(zeros_hbm, acc_sh.at[pl.ds(s * rpt, rpt)])
        plsc.subcore_barrier()
        base = (c * NS + s) * (e // NW)

        def step(i, carry):
            off = base + i * chunk
            pltpu.sync_copy(src_hbm.at[pl.ds(off, chunk)], src_v)
            pltpu.sync_copy(dst_hbm.at[pl.ds(off, chunk)], dst_v)
            pltpu.async_copy(h_hbm.at[src_v], rows_v, sem).wait()
            pltpu.sync_copy(rows_v, acc_sh.at[dst_v], add=True)
            return carry

        lax.fori_loop(0, iters, step, 0)
        plsc.subcore_barrier()
        pltpu.sync_copy(acc_sh.at[pl.ds(s * rpt, rpt)],
                        out_hbm.at[c, pl.ds(s * rpt, rpt)])

    return edge_kernel


def _dense_pre(degp, x, W1, bn):
    n, d = x.shape

    def body(degp_ref, x_ref, w_ref, h_ref, dinv_ref):
        p = degp_ref[...]
        dv = lax.rsqrt(1.0 + p[0, :, :1] + p[1, :, :1])
        h = jnp.dot(x_ref[...], w_ref[...], preferred_element_type=jnp.float32)
        h_ref[...] = h * dv
        dinv_ref[...] = dv

    return pl.pallas_call(
        body,
        grid=(n // bn,),
        in_specs=[
            pl.BlockSpec((NC, bn, 128), lambda i: (0, i, 0)),
            pl.BlockSpec((bn, d), lambda i: (i, 0)),
            pl.BlockSpec((d, d), lambda i: (0, 0)),
        ],
        out_specs=[
            pl.BlockSpec((bn, d), lambda i: (i, 0)),
            pl.BlockSpec((bn, 1), lambda i: (i, 0)),
        ],
        out_shape=[
            jax.ShapeDtypeStruct((n, d), jnp.float32),
            jax.ShapeDtypeStruct((n, 1), jnp.float32),
        ],
    )(degp, x, W1)


def _dense_mid(accp, h1s, dinv, b1, W2, bn):
    n, d = h1s.shape

    def body(accp_ref, h_ref, dinv_ref, b_ref, w_ref, o_ref):
        p = accp_ref[...]
        dv = dinv_ref[...]
        z = (p[0] + p[1] + h_ref[...]) * dv + b_ref[...]
        g = jax.nn.gelu(z)
        o_ref[...] = jnp.dot(g, w_ref[...],
                             preferred_element_type=jnp.float32) * dv

    return pl.pallas_call(
        body,
        grid=(n // bn,),
        in_specs=[
            pl.BlockSpec((NC, bn, d), lambda i: (0, i, 0)),
            pl.BlockSpec((bn, d), lambda i: (i, 0)),
            pl.BlockSpec((bn, 1), lambda i: (i, 0)),
            pl.BlockSpec((1, d), lambda i: (0, 0)),
            pl.BlockSpec((d, d), lambda i: (0, 0)),
        ],
        out_specs=pl.BlockSpec((bn, d), lambda i: (i, 0)),
        out_shape=jax.ShapeDtypeStruct((n, d), jnp.float32),
    )(accp, h1s, dinv, b1, W2)


def _dense_post(accp, h2s, dinv, b2, bn):
    n, d = h2s.shape

    def body(accp_ref, h_ref, dinv_ref, b_ref, o_ref):
        p = accp_ref[...]
        o_ref[...] = (p[0] + p[1] + h_ref[...]) * dinv_ref[...] + b_ref[...]

    return pl.pallas_call(
        body,
        grid=(n // bn,),
        in_specs=[
            pl.BlockSpec((NC, bn, d), lambda i: (0, i, 0)),
            pl.BlockSpec((bn, d), lambda i: (i, 0)),
            pl.BlockSpec((bn, 1), lambda i: (i, 0)),
            pl.BlockSpec((1, d), lambda i: (0, 0)),
        ],
        out_specs=pl.BlockSpec((bn, d), lambda i: (i, 0)),
        out_shape=jax.ShapeDtypeStruct((n, d), jnp.float32),
    )(accp, h2s, dinv, b2)


def kernel(x, edge_index, W1, b1, W2, b2):
    n, d = x.shape
    e = edge_index.shape[1]
    chunk = 80
    bn = 1000
    assert e % (NW * chunk) == 0 and n % NS == 0 and n % bn == 0

    src = edge_index[0].astype(jnp.int32)
    dst = edge_index[1].astype(jnp.int32)
    rpt = _pad_nodes(n) // NS
    zeros16 = jnp.zeros((rpt, 128), jnp.float32)
    ones16 = jnp.ones((chunk, 128), jnp.float32)
    zerosd = jnp.zeros((rpt, d), jnp.float32)
    b1r = b1.reshape(1, d)
    b2r = b2.reshape(1, d)

    edge_k = _make_edge_kernel(n, d, e, chunk)

    degp = _make_deg_kernel(n, e, chunk)(dst, zeros16, ones16)
    h1s, dinv = _dense_pre(degp, x, W1, bn)
    acc1 = edge_k(h1s, src, dst, zerosd)
    h2s = _dense_mid(acc1, h1s, dinv, b1r, W2, bn)
    acc2 = edge_k(h2s, src, dst, zerosd)
    return _dense_post(acc2, h2s, dinv, b2r, bn)
